# Initial kernel scaffold; baseline (speedup 1.0000x reference)
import jax
import jax.numpy as jnp
from jax.experimental import pallas as pl


def kernel(x, graph_size_list):
    raise NotImplementedError("write your pallas kernel here")


# speedup vs baseline: 49.7106x; 49.7106x over previous
import jax
import jax.numpy as jnp
import numpy as np
from jax import lax
from jax.experimental import pallas as pl
from jax.experimental.pallas import tpu as pltpu
from jax.experimental.pallas import tpu_sc as plsc

NSEG = 256
NW = 32
SEG_PER_W = NSEG // NW
L = 16
NEG = np.float32(-3e38)
N_BISECT = 24

_FLAT_IDX = np.concatenate(
    [256 * i + np.arange(i) for i in range(NSEG)]).astype(np.int32)


def _tec_body(x_hbm, gsl_hbm, out_hbm, win, outv, gslv):
    wid = lax.axis_index("s") * 2 + lax.axis_index("c")
    pltpu.sync_copy(gsl_hbm, gslv)
    lanes = lax.iota(jnp.int32, L)
    for j in range(SEG_PER_W):
        seg = wid + NW * j
        n = seg
        nmax = (NW - 1) + NW * j
        LEN = ((7 + nmax + 7) // 8) * 8
        C = (nmax + 15) // 16
        off = (n * (n - 1)) // 2
        start8 = pl.multiple_of((off // 8) * 8, 8)
        shift = off - start8
        pltpu.sync_copy(x_hbm.at[pl.ds(start8, LEN)], win.at[pl.ds(0, LEN)])

        zs = []
        for c in range(C):
            v = win[pl.ds(shift + 16 * c, 16)]
            pos = lanes + 16 * c
            zs.append(jnp.where(pos < n, v, NEG))

        def bcast(v):
            return lax.broadcast_in_dim(v, (L,), ())

        m = zs[0]
        for c in range(1, C):
            m = jnp.maximum(m, zs[c])
        zmaxv = bcast(jnp.max(m))
        n_fv = jnp.maximum(bcast(n).astype(jnp.float32), 1.0)
        lo = zmaxv - 1.0
        hi = zmaxv - 1.0 / n_fv

        def bis(_, carry):
            lo, hi = carry
            mid = 0.5 * (lo + hi)
            acc = jnp.maximum(zs[0] - mid, 0.0)
            for c in range(1, C):
                acc = acc + jnp.maximum(zs[c] - mid, 0.0)
            big = bcast(jnp.sum(acc)) > 1.0
            return (jnp.where(big, mid, lo), jnp.where(big, hi, mid))

        lo, hi = lax.fori_loop(0, N_BISECT, bis, (lo, hi))
        tau0 = 0.5 * (lo + hi)

        cnt = jnp.zeros((L,), jnp.float32)
        ssum = jnp.zeros((L,), jnp.float32)
        for c in range(C):
            msk = zs[c] > tau0
            cnt = cnt + jnp.where(msk, 1.0, 0.0)
            ssum = ssum + jnp.where(msk, zs[c], 0.0)
        tau = ((bcast(jnp.sum(ssum)) - 1.0) /
               jnp.maximum(bcast(jnp.sum(cnt)), 1.0))

        multv = plsc.load_gather(
            gslv, [jnp.zeros((L,), jnp.int32) + seg]).astype(jnp.float32)
        for c in range(C):
            outv[pl.ds(16 * c, 16)] = jnp.maximum(zs[c] - tau, 0.0) * multv
        base = pl.multiple_of(seg * 256, 256)
        pltpu.sync_copy(outv.at[pl.ds(0, 16 * C)],
                        out_hbm.at[pl.ds(base, 16 * C)])


def kernel(x, graph_size_list):
    x_pad = jnp.pad(x, (0, 16))
    mesh = plsc.VectorSubcoreMesh(core_axis_name="c", subcore_axis_name="s")
    launch = pl.kernel(
        _tec_body,
        mesh=mesh,
        compiler_params=pltpu.CompilerParams(needs_layout_passes=False),
        out_type=jax.ShapeDtypeStruct((NSEG * 256,), jnp.float32),
        scratch_types=[
            pltpu.VMEM((272,), jnp.float32),
            pltpu.VMEM((256,), jnp.float32),
            pltpu.VMEM((256,), jnp.int32),
        ],
    )
    dense = launch(x_pad, graph_size_list)
    return dense[jnp.asarray(_FLAT_IDX)]


# speedup vs baseline: 59.1759x; 1.1904x over previous
import jax
import jax.numpy as jnp
import numpy as np
from jax import lax
from jax.experimental import pallas as pl
from jax.experimental.pallas import tpu as pltpu
from jax.experimental.pallas import tpu_sc as plsc

NSEG = 256
NW = 32
SEG_PER_W = NSEG // NW
L = 16
NEG = np.float32(-3e38)
N_BISECT = 24
WSLOT = 272

_NMAX = [(NW - 1) + NW * j for j in range(SEG_PER_W)]
_LEN = [((7 + nm + 7) // 8) * 8 for nm in _NMAX]
_C = [(nm + 15) // 16 for nm in _NMAX]
_ZOFF = np.concatenate([[0], np.cumsum([16 * c for c in _C])]).astype(int)
_ZTOT = int(_ZOFF[-1])

_FLAT_IDX = np.concatenate(
    [256 * i + np.arange(i) for i in range(NSEG)]).astype(np.int32)


def _tec_body(x_hbm, gsl_hbm, out_hbm, win, zbuf, outv, gslv, dsem):
    wid = lax.axis_index("s") * 2 + lax.axis_index("c")
    lanes = lax.iota(jnp.int32, L)

    def bcast(v):
        return lax.broadcast_in_dim(v, (L,), ())

    ns, shifts, copies = [], [], []
    copies.append(pltpu.async_copy(gsl_hbm, gslv, dsem))
    for j in range(SEG_PER_W):
        n = wid + NW * j
        off = (n * (n - 1)) // 2
        start8 = pl.multiple_of((off // 8) * 8, 8)
        ns.append(n)
        shifts.append(off - start8)
        copies.append(pltpu.async_copy(
            x_hbm.at[pl.ds(start8, _LEN[j])],
            win.at[pl.ds(j * WSLOT, _LEN[j])], dsem))
    for cp in copies:
        cp.wait()

    lo, hi = [], []
    for j in range(SEG_PER_W):
        n, shift = ns[j], shifts[j]
        m = None
        for c in range(_C[j]):
            v = win[pl.ds(j * WSLOT + shift + 16 * c, 16)]
            pos = lanes + 16 * c
            z = jnp.where(pos < n, v, NEG)
            zbuf[pl.ds(int(_ZOFF[j]) + 16 * c, 16)] = z
            m = z if m is None else jnp.maximum(m, z)
        zmaxv = bcast(jnp.max(m))
        n_fv = jnp.maximum(bcast(n).astype(jnp.float32), 1.0)
        lo.append(zmaxv - 1.0)
        hi.append(zmaxv - 1.0 / n_fv)

    def bis(_, carry):
        los, his = carry
        nlos, nhis = [], []
        for j in range(SEG_PER_W):
            mid = 0.5 * (los[j] + his[j])
            acc = None
            for c in range(_C[j]):
                z = zbuf[pl.ds(int(_ZOFF[j]) + 16 * c, 16)]
                r = jnp.maximum(z - mid, 0.0)
                acc = r if acc is None else acc + r
            big = bcast(jnp.sum(acc)) > 1.0
            nlos.append(jnp.where(big, mid, los[j]))
            nhis.append(jnp.where(big, his[j], mid))
        return (tuple(nlos), tuple(nhis))

    lo, hi = lax.fori_loop(0, N_BISECT, bis, (tuple(lo), tuple(hi)))

    out_copies = []
    for j in range(SEG_PER_W):
        tau0 = 0.5 * (lo[j] + hi[j])
        cnt = None
        ssum = None
        zs = []
        for c in range(_C[j]):
            z = zbuf[pl.ds(int(_ZOFF[j]) + 16 * c, 16)]
            zs.append(z)
            msk = z > tau0
            c1 = jnp.where(msk, 1.0, 0.0)
            s1 = jnp.where(msk, z, 0.0)
            cnt = c1 if cnt is None else cnt + c1
            ssum = s1 if ssum is None else ssum + s1
        tau = ((bcast(jnp.sum(ssum)) - 1.0) /
               jnp.maximum(bcast(jnp.sum(cnt)), 1.0))
        seg = ns[j]
        multv = plsc.load_gather(
            gslv, [jnp.zeros((L,), jnp.int32) + seg]).astype(jnp.float32)
        for c in range(_C[j]):
            outv[pl.ds(int(_ZOFF[j]) + 16 * c, 16)] = (
                jnp.maximum(zs[c] - tau, 0.0) * multv)
        base = pl.multiple_of(seg * 256, 256)
        out_copies.append(pltpu.async_copy(
            outv.at[pl.ds(int(_ZOFF[j]), 16 * _C[j])],
            out_hbm.at[pl.ds(base, 16 * _C[j])], dsem))
    for cp in out_copies:
        cp.wait()


def kernel(x, graph_size_list):
    x_pad = jnp.pad(x, (0, 16))
    mesh = plsc.VectorSubcoreMesh(core_axis_name="c", subcore_axis_name="s")
    launch = pl.kernel(
        _tec_body,
        mesh=mesh,
        compiler_params=pltpu.CompilerParams(needs_layout_passes=False),
        out_type=jax.ShapeDtypeStruct((NSEG * 256,), jnp.float32),
        scratch_types=[
            pltpu.VMEM((SEG_PER_W * WSLOT,), jnp.float32),
            pltpu.VMEM((_ZTOT,), jnp.float32),
            pltpu.VMEM((_ZTOT,), jnp.float32),
            pltpu.VMEM((256,), jnp.int32),
            pltpu.SemaphoreType.DMA,
        ],
    )
    dense = launch(x_pad, graph_size_list)
    return dense[jnp.asarray(_FLAT_IDX)]
